# baseline (speedup 1.0000x reference)
import functools

import jax
import jax.numpy as jnp
from jax import lax
from jax.experimental import pallas as pl
from jax.experimental.pallas import tpu as pltpu
from jax.experimental.pallas import tpu_sc as plsc

K = 8192
D = 256
N = 8192
BETA = 0.25
BK = 1024
KT = K // BK
BNO = 512
NTO = N // BNO


def _argmin_kernel(flat_ref, emb_ref, idx_ref, iota_ref):
    k = pl.program_id(0)

    @pl.when(k == 0)
    def _init():
        f = flat_ref[:]
        idx_ref[:, 2 * KT:2 * KT + 1] = jnp.sum(f * f, axis=1, keepdims=True)
        iota_ref[:] = jax.lax.broadcasted_iota(
            jnp.int32, (1, BK), 1).astype(jnp.float32)

    e = emb_ref[:]
    mm2 = jax.lax.dot_general(flat_ref[:].astype(jnp.bfloat16),
                              (e * -2.0).astype(jnp.bfloat16),
                              (((1,), (1,)), ((), ())),
                              preferred_element_type=jnp.float32)
    e2 = jnp.sum(e * e, axis=1)[None, :]
    dist = (idx_ref[:, 2 * KT:2 * KT + 1] + mm2) + e2
    tmin = jnp.min(dist, axis=1, keepdims=True)
    kbase = (k * BK).astype(jnp.float32)
    tidx = jnp.min(jnp.where(dist == tmin, iota_ref[:], float(K)), axis=1,
                   keepdims=True) + kbase

    for kk in range(KT):
        @pl.when(k == kk)
        def _store(kk=kk):
            idx_ref[:, kk:kk + 1] = tmin
            idx_ref[:, KT + kk:KT + kk + 1] = tidx


def _combine_kernel(acc_ref, idx_ref):
    h = KT // 2
    v0 = acc_ref[:, 0:h]
    v1 = acc_ref[:, h:KT]
    i0 = acc_ref[:, KT:KT + h]
    i1 = acc_ref[:, KT + h:2 * KT]
    m0 = jnp.min(v0, axis=1, keepdims=True)
    m1 = jnp.min(v1, axis=1, keepdims=True)
    a0 = jnp.min(jnp.where(v0 == m0, i0, float(K)), axis=1, keepdims=True)
    a1 = jnp.min(jnp.where(v1 == m1, i1, float(K)), axis=1, keepdims=True)
    m0_bf16 = m0.astype(jnp.bfloat16).astype(jnp.float32)
    take1 = m1 < m0_bf16
    idx_ref[:] = jnp.where(take1, a1, a0).astype(jnp.int32)


def _onehot_kernel(idx_ref, oh_ref, counts_ref, csum_ref, iota_ref):
    r = pl.program_id(0)

    @pl.when(r == 0)
    def _iinit():
        iota_ref[:] = jax.lax.broadcasted_iota(jnp.int32, (1, K), 1)

    oh = (iota_ref[:] == idx_ref[:]).astype(jnp.float32)
    oh_ref[:] = oh

    @pl.when(r == 0)
    def _init():
        csum_ref[:] = jnp.zeros((1, K), jnp.float32)

    csum_ref[:] = csum_ref[:] + jnp.sum(oh, axis=0, keepdims=True)

    @pl.when(r == NTO - 1)
    def _emit():
        counts_ref[:] = csum_ref[:]


_SC_INFO = plsc.get_sparse_core_info()
_NW = _SC_INFO.num_cores * _SC_INFO.num_subcores
_BPW = N // _NW
_CH = 64


def _zq_gather_kernel(emb_hbm, idx_hbm, out_hbm, idx_v, rows_v, sem):
    wid = lax.axis_index("s") * _SC_INFO.num_cores + lax.axis_index("c")
    base = wid * _BPW
    for c in range(_BPW // _CH):
        off = base + c * _CH
        pltpu.sync_copy(idx_hbm.at[pl.ds(off, _CH)], idx_v)
        pltpu.async_copy(emb_hbm.at[idx_v], rows_v, sem).wait()
        pltpu.sync_copy(rows_v, out_hbm.at[pl.ds(off, _CH)])


def _finish_kernel(flat_ref, zq_ref, counts_ref, loss_ref, ppx_ref, zqout_ref):
    f = flat_ref[:]
    q = zq_ref[:]
    d = q - f
    m = jnp.mean(d * d)
    loss_ref[:] = (m + BETA * m).reshape(1, 1)
    zqout_ref[:] = f + (q - f)
    p = counts_ref[:] * (1.0 / N)
    ent = jnp.sum(p * jnp.log(p + 1e-10))
    ppx_ref[:] = jnp.exp(-ent).reshape(1, 1)


@functools.partial(jax.jit, static_argnames=())
def kernel(latents, embedding):
    lat = jnp.transpose(latents, (0, 2, 3, 1))
    flat = lat.reshape(-1, D)

    tiles = pl.pallas_call(
        _argmin_kernel,
        grid=(KT,),
        in_specs=[
            pl.BlockSpec((N, D), lambda k: (0, 0)),
            pl.BlockSpec((BK, D), lambda k: (k, 0)),
        ],
        out_specs=pl.BlockSpec((N, 2 * KT + 1), lambda k: (0, 0)),
        out_shape=jax.ShapeDtypeStruct((N, 2 * KT + 1), jnp.float32),
        scratch_shapes=[
            pltpu.VMEM((1, BK), jnp.float32),
        ],
    )(flat, embedding)

    idx = pl.pallas_call(
        _combine_kernel,
        in_specs=[pl.BlockSpec((N, 2 * KT + 1), lambda: (0, 0))],
        out_specs=pl.BlockSpec((N, 1), lambda: (0, 0)),
        out_shape=jax.ShapeDtypeStruct((N, 1), jnp.int32),
    )(tiles)

    min_embed, counts = pl.pallas_call(
        _onehot_kernel,
        grid=(NTO,),
        in_specs=[
            pl.BlockSpec((BNO, 1), lambda r: (r, 0)),
        ],
        out_specs=[
            pl.BlockSpec((BNO, K), lambda r: (r, 0)),
            pl.BlockSpec((1, K), lambda r: (0, 0)),
        ],
        out_shape=[
            jax.ShapeDtypeStruct((N, K), jnp.float32),
            jax.ShapeDtypeStruct((1, K), jnp.float32),
        ],
        scratch_shapes=[pltpu.VMEM((1, K), jnp.float32),
                        pltpu.VMEM((1, K), jnp.int32)],
    )(idx)

    zq = pl.kernel(
        _zq_gather_kernel,
        mesh=plsc.VectorSubcoreMesh(core_axis_name="c", subcore_axis_name="s"),
        out_type=jax.ShapeDtypeStruct((N, D), jnp.float32),
        scratch_types=[
            pltpu.VMEM((_CH,), jnp.int32),
            pltpu.VMEM((_CH, D), jnp.float32),
            pltpu.SemaphoreType.DMA,
        ],
    )(embedding, idx.reshape(N))

    loss, ppx, zq_out = pl.pallas_call(
        _finish_kernel,
        in_specs=[
            pl.BlockSpec((N, D), lambda: (0, 0)),
            pl.BlockSpec((N, D), lambda: (0, 0)),
            pl.BlockSpec((1, K), lambda: (0, 0)),
        ],
        out_specs=[
            pl.BlockSpec((1, 1), lambda: (0, 0)),
            pl.BlockSpec((1, 1), lambda: (0, 0)),
            pl.BlockSpec((N, D), lambda: (0, 0)),
        ],
        out_shape=[
            jax.ShapeDtypeStruct((1, 1), jnp.float32),
            jax.ShapeDtypeStruct((1, 1), jnp.float32),
            jax.ShapeDtypeStruct((N, D), jnp.float32),
        ],
    )(flat, zq, counts)

    z_q = jnp.transpose(zq_out.reshape(lat.shape), (0, 3, 1, 2))
    return (loss.reshape(()), z_q, ppx.reshape(()), min_embed, idx)


# speedup vs baseline: 1.0046x; 1.0046x over previous
import functools

import jax
import jax.numpy as jnp
from jax import lax
from jax.experimental import pallas as pl
from jax.experimental.pallas import tpu as pltpu
from jax.experimental.pallas import tpu_sc as plsc

K = 8192
D = 256
N = 8192
BETA = 0.25
BK = 1024
KT = K // BK
BNO = 256
NTO = N // BNO


def _argmin_kernel(flat_ref, emb_ref, idx_ref, iota_ref):
    k = pl.program_id(0)

    @pl.when(k == 0)
    def _init():
        f = flat_ref[:]
        idx_ref[:, 2 * KT:2 * KT + 1] = jnp.sum(f * f, axis=1, keepdims=True)
        iota_ref[:] = jax.lax.broadcasted_iota(
            jnp.int32, (1, BK), 1).astype(jnp.float32)

    e = emb_ref[:]
    mm2 = jax.lax.dot_general(flat_ref[:].astype(jnp.bfloat16),
                              (e * -2.0).astype(jnp.bfloat16),
                              (((1,), (1,)), ((), ())),
                              preferred_element_type=jnp.float32)
    e2 = jnp.sum(e * e, axis=1)[None, :]
    dist = (idx_ref[:, 2 * KT:2 * KT + 1] + mm2) + e2
    tmin = jnp.min(dist, axis=1, keepdims=True)
    kbase = (k * BK).astype(jnp.float32)
    tidx = jnp.min(jnp.where(dist == tmin, iota_ref[:], float(K)), axis=1,
                   keepdims=True) + kbase

    for kk in range(KT):
        @pl.when(k == kk)
        def _store(kk=kk):
            idx_ref[:, kk:kk + 1] = tmin
            idx_ref[:, KT + kk:KT + kk + 1] = tidx


def _combine_kernel(acc_ref, idx_ref):
    h = KT // 2
    v0 = acc_ref[:, 0:h]
    v1 = acc_ref[:, h:KT]
    i0 = acc_ref[:, KT:KT + h]
    i1 = acc_ref[:, KT + h:2 * KT]
    m0 = jnp.min(v0, axis=1, keepdims=True)
    m1 = jnp.min(v1, axis=1, keepdims=True)
    a0 = jnp.min(jnp.where(v0 == m0, i0, float(K)), axis=1, keepdims=True)
    a1 = jnp.min(jnp.where(v1 == m1, i1, float(K)), axis=1, keepdims=True)
    m0_bf16 = m0.astype(jnp.bfloat16).astype(jnp.float32)
    take1 = m1 < m0_bf16
    idx_ref[:] = jnp.where(take1, a1, a0).astype(jnp.int32)


def _onehot_kernel(idx_ref, oh_ref, counts_ref, csum_ref, iota_ref):
    r = pl.program_id(0)

    @pl.when(r == 0)
    def _iinit():
        iota_ref[:] = jax.lax.broadcasted_iota(jnp.int32, (1, K), 1)

    oh = (iota_ref[:] == idx_ref[:]).astype(jnp.float32)
    oh_ref[:] = oh

    @pl.when(r == 0)
    def _init():
        csum_ref[:] = jnp.zeros((1, K), jnp.float32)

    csum_ref[:] = csum_ref[:] + jnp.sum(oh, axis=0, keepdims=True)

    @pl.when(r == NTO - 1)
    def _emit():
        counts_ref[:] = csum_ref[:]


_SC_INFO = plsc.get_sparse_core_info()
_NW = _SC_INFO.num_cores * _SC_INFO.num_subcores
_BPW = N // _NW
_CH = 64


def _zq_gather_kernel(emb_hbm, idx_hbm, out_hbm, idx_v, rows_v, sem):
    wid = lax.axis_index("s") * _SC_INFO.num_cores + lax.axis_index("c")
    base = wid * _BPW
    for c in range(_BPW // _CH):
        off = base + c * _CH
        pltpu.sync_copy(idx_hbm.at[pl.ds(off, _CH)], idx_v)
        pltpu.async_copy(emb_hbm.at[idx_v], rows_v, sem).wait()
        pltpu.sync_copy(rows_v, out_hbm.at[pl.ds(off, _CH)])


def _finish_kernel(flat_ref, zq_ref, counts_ref, loss_ref, ppx_ref, zqout_ref):
    f = flat_ref[:]
    q = zq_ref[:]
    d = q - f
    m = jnp.mean(d * d)
    loss_ref[:] = (m + BETA * m).reshape(1, 1)
    zqout_ref[:] = f + (q - f)
    p = counts_ref[:] * (1.0 / N)
    ent = jnp.sum(p * jnp.log(p + 1e-10))
    ppx_ref[:] = jnp.exp(-ent).reshape(1, 1)


@functools.partial(jax.jit, static_argnames=())
def kernel(latents, embedding):
    lat = jnp.transpose(latents, (0, 2, 3, 1))
    flat = lat.reshape(-1, D)

    tiles = pl.pallas_call(
        _argmin_kernel,
        grid=(KT,),
        in_specs=[
            pl.BlockSpec((N, D), lambda k: (0, 0)),
            pl.BlockSpec((BK, D), lambda k: (k, 0)),
        ],
        out_specs=pl.BlockSpec((N, 2 * KT + 1), lambda k: (0, 0)),
        out_shape=jax.ShapeDtypeStruct((N, 2 * KT + 1), jnp.float32),
        scratch_shapes=[
            pltpu.VMEM((1, BK), jnp.float32),
        ],
    )(flat, embedding)

    idx = pl.pallas_call(
        _combine_kernel,
        in_specs=[pl.BlockSpec((N, 2 * KT + 1), lambda: (0, 0))],
        out_specs=pl.BlockSpec((N, 1), lambda: (0, 0)),
        out_shape=jax.ShapeDtypeStruct((N, 1), jnp.int32),
    )(tiles)

    min_embed, counts = pl.pallas_call(
        _onehot_kernel,
        grid=(NTO,),
        in_specs=[
            pl.BlockSpec((BNO, 1), lambda r: (r, 0)),
        ],
        out_specs=[
            pl.BlockSpec((BNO, K), lambda r: (r, 0)),
            pl.BlockSpec((1, K), lambda r: (0, 0)),
        ],
        out_shape=[
            jax.ShapeDtypeStruct((N, K), jnp.float32),
            jax.ShapeDtypeStruct((1, K), jnp.float32),
        ],
        scratch_shapes=[pltpu.VMEM((1, K), jnp.float32),
                        pltpu.VMEM((1, K), jnp.int32)],
    )(idx)

    zq = pl.kernel(
        _zq_gather_kernel,
        mesh=plsc.VectorSubcoreMesh(core_axis_name="c", subcore_axis_name="s"),
        out_type=jax.ShapeDtypeStruct((N, D), jnp.float32),
        scratch_types=[
            pltpu.VMEM((_CH,), jnp.int32),
            pltpu.VMEM((_CH, D), jnp.float32),
            pltpu.SemaphoreType.DMA,
        ],
    )(embedding, idx.reshape(N))

    loss, ppx, zq_out = pl.pallas_call(
        _finish_kernel,
        in_specs=[
            pl.BlockSpec((N, D), lambda: (0, 0)),
            pl.BlockSpec((N, D), lambda: (0, 0)),
            pl.BlockSpec((1, K), lambda: (0, 0)),
        ],
        out_specs=[
            pl.BlockSpec((1, 1), lambda: (0, 0)),
            pl.BlockSpec((1, 1), lambda: (0, 0)),
            pl.BlockSpec((N, D), lambda: (0, 0)),
        ],
        out_shape=[
            jax.ShapeDtypeStruct((1, 1), jnp.float32),
            jax.ShapeDtypeStruct((1, 1), jnp.float32),
            jax.ShapeDtypeStruct((N, D), jnp.float32),
        ],
    )(flat, zq, counts)

    z_q = jnp.transpose(zq_out.reshape(lat.shape), (0, 3, 1, 2))
    return (loss.reshape(()), z_q, ppx.reshape(()), min_embed, idx)
